# baseline (speedup 1.0000x reference)
import jax
import jax.numpy as jnp
from jax import lax
from jax.experimental import pallas as pl
from jax.experimental.pallas import tpu as pltpu
from jax.experimental.pallas import tpu_sc as plsc

_E = 8
_K = 2
_TB = 512
_BLK = 256
_NTILES = 32
_L = 16
_NBE = 64


def _mm_t(a, b):
    return lax.dot_general(a, b, (((1,), (1,)), ((), ())),
                           preferred_element_type=jnp.float32)


def _mm_t16(a, b):
    return _mm_t(a.astype(jnp.bfloat16), b.astype(jnp.bfloat16))


def _shared_body(x_ref, sg_ref, su_ref, sd_ref, xs_ref):
    x = x_ref[...]
    g = _mm_t16(x, sg_ref[...])
    u = _mm_t16(x, su_ref[...])
    h = jax.nn.sigmoid(g) * u
    xs_ref[...] = x + _mm_t16(h, sd_ref[...])


def _router_body(x_ref, r_ref, egw_ref, euw_ref, edw_ref,
                 mt_ref, pst_ref, beo_ref, eg16_ref, eu16_ref, ed16_ref,
                 carry_ref):
    t = pl.program_id(0)
    nt = pl.num_programs(0)

    @pl.when(t == 0)
    def _init():
        carry_ref[...] = jnp.zeros_like(carry_ref)

    eg16_ref[...] = egw_ref[...].astype(jnp.bfloat16)
    eu16_ref[...] = euw_ref[...].astype(jnp.bfloat16)
    ed16_ref[...] = edw_ref[...].astype(jnp.bfloat16)
    x = x_ref[...]
    logits = jnp.dot(x, r_ref[...], preferred_element_type=jnp.float32)
    m = jnp.max(logits, axis=-1, keepdims=True)
    ex = jnp.exp(logits - m)
    sm = ex / jnp.sum(ex, axis=-1, keepdims=True)
    ids = jax.lax.broadcasted_iota(jnp.int32, sm.shape, 1)
    m1 = jnp.max(sm, axis=-1, keepdims=True)
    i1 = jnp.min(jnp.where(sm == m1, ids, _E), axis=-1, keepdims=True)
    s2 = jnp.where(ids == i1, -jnp.inf, sm)
    m2 = jnp.max(s2, axis=-1, keepdims=True)
    i2 = jnp.min(jnp.where(s2 == m2, ids, _E), axis=-1, keepdims=True)

    oh0 = (ids == i1).astype(jnp.float32)
    oh1 = (ids == i2).astype(jnp.float32)
    oh = oh0 + oh1
    row = jax.lax.broadcasted_iota(jnp.int32, (_TB, _TB), 0)
    col = jax.lax.broadcasted_iota(jnp.int32, (_TB, _TB), 1)
    tril = (row > col).astype(jnp.float32)
    c = jnp.dot(tril, oh, preferred_element_type=jnp.float32) + carry_ref[...]
    r0 = jnp.sum(c * oh0, axis=-1, keepdims=True)
    r1 = jnp.sum(c * oh1, axis=-1, keepdims=True)
    carry_new = carry_ref[...] + jnp.sum(oh, axis=0, keepdims=True)
    carry_ref[...] = carry_new

    lane = jax.lax.broadcasted_iota(jnp.int32, (_TB, 8), 1)
    meta = jnp.where(
        lane == 0, m1,
        jnp.where(lane == 1, m2,
                  jnp.where(lane == 2, i1.astype(jnp.float32),
                            jnp.where(lane == 3, i2.astype(jnp.float32),
                                      jnp.where(lane == 4, r0,
                                                jnp.where(lane == 5, r1,
                                                          0.0))))))
    eye = (row == col).astype(jnp.float32)
    mt_ref[...] = lax.dot_general(
        meta, eye, (((0,), (0,)), ((), ())),
        precision=lax.Precision.HIGHEST,
        preferred_element_type=jnp.float32)[None]

    @pl.when(t == nt - 1)
    def _finish():
        cntv = carry_new
        bc = jnp.floor((cntv + (_BLK - 1)) * (1.0 / _BLK))
        erow = jax.lax.broadcasted_iota(jnp.int32, (_E, _E), 0)
        ecol = jax.lax.broadcasted_iota(jnp.int32, (_E, _E), 1)
        lower = (erow <= ecol).astype(jnp.float32)
        cum = jnp.dot(bc, lower, preferred_element_type=jnp.float32)
        excl = cum - bc
        pstv = jnp.concatenate(
            [excl * _BLK, cum[:, 7:8], jnp.zeros((1, 7), jnp.float32)],
            axis=1)
        pst_ref[...] = jnp.broadcast_to(pstv, (8, 16)).astype(jnp.int32)
        bvec = jax.lax.broadcasted_iota(
            jnp.int32, (1, _NBE), 1).astype(jnp.float32)
        acc = jnp.zeros((1, _NBE), jnp.float32)
        for e in range(_E):
            acc = acc + (cum[0, e] <= bvec).astype(jnp.float32)
        acc = jnp.clip(acc, 0, _E - 1)
        beo_ref[...] = jnp.broadcast_to(acc, (8, _NBE)).astype(jnp.int32)


def _scatter_body(x2, mt, pst, xg, sp0, sp1,
                  iv0, iv1, rv0, rv1, psv, spf0, spf1, sp2d0, sp2d1,
                  xbuf, sem):
    chunk = 4096 // _NTILES
    nsub = chunk // 32
    w = lax.axis_index("s") * 2 + lax.axis_index("c")
    t0 = w * chunk
    n_i = t0 // _TB
    off = t0 % _TB
    pltpu.sync_copy(mt.at[n_i, 2, pl.ds(off, chunk)], iv0)
    pltpu.sync_copy(mt.at[n_i, 3, pl.ds(off, chunk)], iv1)
    pltpu.sync_copy(mt.at[n_i, 4, pl.ds(off, chunk)], rv0)
    pltpu.sync_copy(mt.at[n_i, 5, pl.ds(off, chunk)], rv1)
    pltpu.sync_copy(pst.at[0], psv)
    for l in range(chunk // _L):
        sl = pl.ds(l * _L, _L)
        i0v = iv0[sl].astype(jnp.int32)
        i1v = iv1[sl].astype(jnp.int32)
        st0 = plsc.load_gather(psv, [i0v])
        st1 = plsc.load_gather(psv, [i1v])
        spv0 = st0 + rv0[sl].astype(jnp.int32)
        spv1 = st1 + rv1[sl].astype(jnp.int32)
        spf0[sl] = spv0
        spf1[sl] = spv1
        sp2d0[l // 2, pl.ds((l % 2) * _L, _L)] = spv0
        sp2d1[l // 2, pl.ds((l % 2) * _L, _L)] = spv1
    pltpu.sync_copy(spf0, sp0.at[pl.ds(t0, chunk)])
    pltpu.sync_copy(spf1, sp1.at[pl.ds(t0, chunk)])
    for j in range(nsub):
        pltpu.sync_copy(x2.at[pl.ds(t0 + j * 32, 32)], xbuf)
        pltpu.async_copy(xbuf, xg.at[sp2d0.at[j]], sem).wait()
        pltpu.async_copy(xbuf, xg.at[sp2d1.at[j]], sem).wait()


def _expert_body(be_ref, xg_ref, eg_ref, eu_ref, ed_ref, yg_ref):
    b = pl.program_id(0)
    nb = pl.num_programs(0)

    @pl.when(b < be_ref[nb])
    def _compute():
        x = xg_ref[...].astype(jnp.bfloat16)
        ge = _mm_t(x, eg_ref[0])
        ue = _mm_t(x, eu_ref[0])
        he = (jax.nn.sigmoid(ge) * ue).astype(jnp.bfloat16)
        yg_ref[...] = _mm_t(he, ed_ref[0])


def _combine_body(xs, yg, sp0, sp1, mt, out,
                  spv0, spv1, wv0, wv1, y0a, y1a, y0b, y1b, xsa, xsb,
                  sem0a, sem1a, sem0b, sem1b, semxa, semxb):
    chunk = 4096 // _NTILES
    sub = 16
    H = 1024
    w = lax.axis_index("s") * 2 + lax.axis_index("c")
    t0 = w * chunk
    n_i = t0 // _TB
    off = t0 % _TB
    pltpu.sync_copy(sp0.at[pl.ds(t0, chunk)], spv0)
    pltpu.sync_copy(sp1.at[pl.ds(t0, chunk)], spv1)
    pltpu.sync_copy(mt.at[n_i, 0, pl.ds(off, chunk)], wv0.at[pl.ds(0, chunk)])
    pltpu.sync_copy(mt.at[n_i, 1, pl.ds(off, chunk)], wv1.at[pl.ds(0, chunk)])
    ybufs = ((y0a, y1a, sem0a, sem1a, semxa), (y0b, y1b, sem0b, sem1b, semxb))
    xbufs = (xsa, xsb)
    nsub = chunk // sub

    def start(j, bank):
        y0, y1, s0, s1, sx = ybufs[bank]
        c0 = pltpu.async_copy(yg.at[spv0.at[pl.ds(j * sub, sub)]], y0, s0)
        c1 = pltpu.async_copy(yg.at[spv1.at[pl.ds(j * sub, sub)]], y1, s1)
        cx = pltpu.async_copy(xs.at[pl.ds(t0 + j * sub, sub)], xbufs[bank],
                              sx)
        return c0, c1, cx

    pend = start(0, 0)
    for j in range(nsub):
        bank = j % 2
        cur = pend
        if j + 1 < nsub:
            pend = start(j + 1, 1 - bank)
        for c in cur:
            c.wait()
        y0, y1 = ybufs[bank][0], ybufs[bank][1]
        xb = xbufs[bank]

        def body(t, _):
            ws0 = wv0[pl.ds(j * sub + t, _L)][0]
            ws1 = wv1[pl.ds(j * sub + t, _L)][0]
            for l in range(H // _L):
                sl = pl.ds(l * _L, _L)
                xb[t, sl] = xb[t, sl] + ws0 * y0[t, sl] + ws1 * y1[t, sl]
            return 0

        lax.fori_loop(0, sub, body, 0)
        pltpu.sync_copy(xb, out.at[pl.ds(t0 + j * sub, sub)])


def kernel(hidden_states, shared_gate_w, shared_up_w, shared_down_w,
           expert_gate_w, expert_up_w, expert_down_w, router_w):
    B, S, H = hidden_states.shape
    I = shared_gate_w.shape[0]
    T = B * S
    x2 = hidden_states.reshape(T, H)

    nt = T // _TB
    nb = (T * _K) // _BLK + _E
    pp = nb * _BLK

    a2_out = pl.pallas_call(
        _router_body,
        grid=(nt,),
        in_specs=[
            pl.BlockSpec((_TB, H), lambda t: (t, 0)),
            pl.BlockSpec((H, _E), lambda t: (0, 0)),
            pl.BlockSpec((1, I, H), lambda t: (t, 0, 0)),
            pl.BlockSpec((1, I, H), lambda t: (t, 0, 0)),
            pl.BlockSpec((1, H, I), lambda t: (t, 0, 0)),
        ],
        out_specs=[
            pl.BlockSpec((1, 8, _TB), lambda t: (t, 0, 0)),
            pl.BlockSpec((8, 16), lambda t: (0, 0)),
            pl.BlockSpec((8, _NBE), lambda t: (0, 0)),
            pl.BlockSpec((1, I, H), lambda t: (t, 0, 0)),
            pl.BlockSpec((1, I, H), lambda t: (t, 0, 0)),
            pl.BlockSpec((1, H, I), lambda t: (t, 0, 0)),
        ],
        out_shape=[
            jax.ShapeDtypeStruct((nt, 8, _TB), jnp.float32),
            jax.ShapeDtypeStruct((8, 16), jnp.int32),
            jax.ShapeDtypeStruct((8, _NBE), jnp.int32),
            jax.ShapeDtypeStruct((_E, I, H), jnp.bfloat16),
            jax.ShapeDtypeStruct((_E, I, H), jnp.bfloat16),
            jax.ShapeDtypeStruct((_E, H, I), jnp.bfloat16),
        ],
        scratch_shapes=[pltpu.VMEM((1, _E), jnp.float32)],
    )(x2, router_w.T, expert_gate_w, expert_up_w, expert_down_w)
    mt, pst, beo, eg16, eu16, ed16 = a2_out
    used = pst[0, 8:9]
    be = jnp.concatenate([beo[0, :nb], used])

    xs = pl.pallas_call(
        _shared_body,
        grid=(nt,),
        in_specs=[
            pl.BlockSpec((_TB, H), lambda t: (t, 0)),
            pl.BlockSpec((I, H), lambda t: (0, 0)),
            pl.BlockSpec((I, H), lambda t: (0, 0)),
            pl.BlockSpec((H, I), lambda t: (0, 0)),
        ],
        out_specs=pl.BlockSpec((_TB, H), lambda t: (t, 0)),
        out_shape=jax.ShapeDtypeStruct((T, H), jnp.float32),
    )(x2, shared_gate_w, shared_up_w, shared_down_w)

    mesh = plsc.VectorSubcoreMesh(core_axis_name="c", subcore_axis_name="s")
    chunk = T // _NTILES
    sc_params = pltpu.CompilerParams(needs_layout_passes=False)
    sc_scatter = pl.kernel(
        _scatter_body,
        compiler_params=sc_params,
        out_type=[
            jax.ShapeDtypeStruct((pp, H), jnp.float32),
            jax.ShapeDtypeStruct((T,), jnp.int32),
            jax.ShapeDtypeStruct((T,), jnp.int32),
        ],
        mesh=mesh,
        scratch_types=[
            pltpu.VMEM((chunk,), jnp.float32),
            pltpu.VMEM((chunk,), jnp.float32),
            pltpu.VMEM((chunk,), jnp.float32),
            pltpu.VMEM((chunk,), jnp.float32),
            pltpu.VMEM((16,), jnp.int32),
            pltpu.VMEM((chunk,), jnp.int32),
            pltpu.VMEM((chunk,), jnp.int32),
            pltpu.VMEM((chunk // 32, 32), jnp.int32),
            pltpu.VMEM((chunk // 32, 32), jnp.int32),
            pltpu.VMEM((32, H), jnp.float32),
            pltpu.SemaphoreType.DMA,
        ],
    )
    xg, sp0, sp1 = sc_scatter(x2, mt, pst)

    yg = pl.pallas_call(
        _expert_body,
        grid_spec=pltpu.PrefetchScalarGridSpec(
            num_scalar_prefetch=1,
            grid=(nb,),
            in_specs=[
                pl.BlockSpec(
                    (_BLK, H),
                    lambda b, be_r: (jnp.minimum(b, be_r[nb] - 1), 0)),
                pl.BlockSpec((1, I, H), lambda b, be_r: (be_r[b], 0, 0)),
                pl.BlockSpec((1, I, H), lambda b, be_r: (be_r[b], 0, 0)),
                pl.BlockSpec((1, H, I), lambda b, be_r: (be_r[b], 0, 0)),
            ],
            out_specs=pl.BlockSpec(
                (_BLK, H),
                lambda b, be_r: (jnp.minimum(b, be_r[nb] - 1), 0)),
        ),
        out_shape=jax.ShapeDtypeStruct((pp, H), jnp.float32),
    )(be, xg, eg16, eu16, ed16)

    sub = 16
    sc_combine = pl.kernel(
        _combine_body,
        compiler_params=sc_params,
        out_type=jax.ShapeDtypeStruct((T, H), jnp.float32),
        mesh=mesh,
        scratch_types=[
            pltpu.VMEM((chunk,), jnp.int32),
            pltpu.VMEM((chunk,), jnp.int32),
            pltpu.VMEM((chunk + _L,), jnp.float32),
            pltpu.VMEM((chunk + _L,), jnp.float32),
            pltpu.VMEM((sub, H), jnp.float32),
            pltpu.VMEM((sub, H), jnp.float32),
            pltpu.VMEM((sub, H), jnp.float32),
            pltpu.VMEM((sub, H), jnp.float32),
            pltpu.VMEM((sub, H), jnp.float32),
            pltpu.VMEM((sub, H), jnp.float32),
            pltpu.SemaphoreType.DMA,
            pltpu.SemaphoreType.DMA,
            pltpu.SemaphoreType.DMA,
            pltpu.SemaphoreType.DMA,
            pltpu.SemaphoreType.DMA,
            pltpu.SemaphoreType.DMA,
        ],
    )
    out2 = sc_combine(xs, yg, sp0, sp1, mt)
    return out2.reshape(B, S, H)


# speedup vs baseline: 1.0429x; 1.0429x over previous
import jax
import jax.numpy as jnp
from jax import lax
from jax.experimental import pallas as pl
from jax.experimental.pallas import tpu as pltpu
from jax.experimental.pallas import tpu_sc as plsc

_E = 8
_K = 2
_TB = 512
_BLK = 256
_NTILES = 32
_L = 16
_NBE = 64


def _mm_t(a, b):
    return lax.dot_general(a, b, (((1,), (1,)), ((), ())),
                           preferred_element_type=jnp.float32)


def _mm_t16(a, b):
    return _mm_t(a.astype(jnp.bfloat16), b.astype(jnp.bfloat16))


def _shared_body(x_ref, sg_ref, su_ref, sd_ref, xs_ref):
    x = x_ref[...]
    g = _mm_t16(x, sg_ref[...])
    u = _mm_t16(x, su_ref[...])
    h = jax.nn.sigmoid(g) * u
    xs_ref[...] = x + _mm_t16(h, sd_ref[...])


def _router_body(x_ref, r_ref, mt_ref, pst_ref, beo_ref, carry_ref):
    t = pl.program_id(0)
    nt = pl.num_programs(0)

    @pl.when(t == 0)
    def _init():
        carry_ref[...] = jnp.zeros_like(carry_ref)

    x = x_ref[...]
    logits = jnp.dot(x, r_ref[...], preferred_element_type=jnp.float32)
    m = jnp.max(logits, axis=-1, keepdims=True)
    ex = jnp.exp(logits - m)
    sm = ex / jnp.sum(ex, axis=-1, keepdims=True)
    ids = jax.lax.broadcasted_iota(jnp.int32, sm.shape, 1)
    m1 = jnp.max(sm, axis=-1, keepdims=True)
    i1 = jnp.min(jnp.where(sm == m1, ids, _E), axis=-1, keepdims=True)
    s2 = jnp.where(ids == i1, -jnp.inf, sm)
    m2 = jnp.max(s2, axis=-1, keepdims=True)
    i2 = jnp.min(jnp.where(s2 == m2, ids, _E), axis=-1, keepdims=True)

    oh0 = (ids == i1).astype(jnp.float32)
    oh1 = (ids == i2).astype(jnp.float32)
    oh = oh0 + oh1
    row = jax.lax.broadcasted_iota(jnp.int32, (_TB, _TB), 0)
    col = jax.lax.broadcasted_iota(jnp.int32, (_TB, _TB), 1)
    tril = (row > col).astype(jnp.float32)
    c = jnp.dot(tril, oh, preferred_element_type=jnp.float32) + carry_ref[...]
    r0 = jnp.sum(c * oh0, axis=-1, keepdims=True)
    r1 = jnp.sum(c * oh1, axis=-1, keepdims=True)
    carry_new = carry_ref[...] + jnp.sum(oh, axis=0, keepdims=True)
    carry_ref[...] = carry_new

    lane = jax.lax.broadcasted_iota(jnp.int32, (_TB, 8), 1)
    meta = jnp.where(
        lane == 0, m1,
        jnp.where(lane == 1, m2,
                  jnp.where(lane == 2, i1.astype(jnp.float32),
                            jnp.where(lane == 3, i2.astype(jnp.float32),
                                      jnp.where(lane == 4, r0,
                                                jnp.where(lane == 5, r1,
                                                          0.0))))))
    eye = (row == col).astype(jnp.float32)
    mt_ref[...] = lax.dot_general(
        meta, eye, (((0,), (0,)), ((), ())),
        precision=lax.Precision.HIGHEST,
        preferred_element_type=jnp.float32)[None]

    @pl.when(t == nt - 1)
    def _finish():
        cntv = carry_new
        bc = jnp.floor((cntv + (_BLK - 1)) * (1.0 / _BLK))
        erow = jax.lax.broadcasted_iota(jnp.int32, (_E, _E), 0)
        ecol = jax.lax.broadcasted_iota(jnp.int32, (_E, _E), 1)
        lower = (erow <= ecol).astype(jnp.float32)
        cum = jnp.dot(bc, lower, preferred_element_type=jnp.float32)
        excl = cum - bc
        pstv = jnp.concatenate(
            [excl * _BLK, cum[:, 7:8], jnp.zeros((1, 7), jnp.float32)],
            axis=1)
        pst_ref[...] = jnp.broadcast_to(pstv, (8, 16)).astype(jnp.int32)
        bvec = jax.lax.broadcasted_iota(
            jnp.int32, (1, _NBE), 1).astype(jnp.float32)
        acc = jnp.zeros((1, _NBE), jnp.float32)
        for e in range(_E):
            acc = acc + (cum[0, e] <= bvec).astype(jnp.float32)
        acc = jnp.clip(acc, 0, _E - 1)
        beo_ref[...] = jnp.broadcast_to(acc, (8, _NBE)).astype(jnp.int32)


def _scatter_body(x2, mt, pst, xg, sp0, sp1,
                  iv0, iv1, rv0, rv1, psv, spf0, spf1, sp2d0, sp2d1,
                  xbuf, sem):
    chunk = 4096 // _NTILES
    nsub = chunk // 32
    w = lax.axis_index("s") * 2 + lax.axis_index("c")
    t0 = w * chunk
    n_i = t0 // _TB
    off = t0 % _TB
    pltpu.sync_copy(mt.at[n_i, 2, pl.ds(off, chunk)], iv0)
    pltpu.sync_copy(mt.at[n_i, 3, pl.ds(off, chunk)], iv1)
    pltpu.sync_copy(mt.at[n_i, 4, pl.ds(off, chunk)], rv0)
    pltpu.sync_copy(mt.at[n_i, 5, pl.ds(off, chunk)], rv1)
    pltpu.sync_copy(pst.at[0], psv)
    for l in range(chunk // _L):
        sl = pl.ds(l * _L, _L)
        i0v = iv0[sl].astype(jnp.int32)
        i1v = iv1[sl].astype(jnp.int32)
        st0 = plsc.load_gather(psv, [i0v])
        st1 = plsc.load_gather(psv, [i1v])
        spv0 = st0 + rv0[sl].astype(jnp.int32)
        spv1 = st1 + rv1[sl].astype(jnp.int32)
        spf0[sl] = spv0
        spf1[sl] = spv1
        sp2d0[l // 2, pl.ds((l % 2) * _L, _L)] = spv0
        sp2d1[l // 2, pl.ds((l % 2) * _L, _L)] = spv1
    pltpu.sync_copy(spf0, sp0.at[pl.ds(t0, chunk)])
    pltpu.sync_copy(spf1, sp1.at[pl.ds(t0, chunk)])
    for j in range(nsub):
        pltpu.sync_copy(x2.at[pl.ds(t0 + j * 32, 32)], xbuf)
        pltpu.async_copy(xbuf, xg.at[sp2d0.at[j]], sem).wait()
        pltpu.async_copy(xbuf, xg.at[sp2d1.at[j]], sem).wait()


def _expert_body(be_ref, xg_ref, eg_ref, eu_ref, ed_ref, yg_ref):
    b = pl.program_id(0)
    nb = pl.num_programs(0)

    @pl.when(b < be_ref[nb])
    def _compute():
        x = xg_ref[...]
        ge = _mm_t16(x, eg_ref[0])
        ue = _mm_t16(x, eu_ref[0])
        he = jax.nn.sigmoid(ge) * ue
        yg_ref[...] = _mm_t16(he, ed_ref[0])


def _combine_body(xs, yg, sp0, sp1, mt, out,
                  spv0, spv1, wv0, wv1, y0a, y1a, y0b, y1b, xsa, xsb,
                  sem0a, sem1a, sem0b, sem1b, semxa, semxb):
    chunk = 4096 // _NTILES
    sub = 16
    H = 1024
    w = lax.axis_index("s") * 2 + lax.axis_index("c")
    t0 = w * chunk
    n_i = t0 // _TB
    off = t0 % _TB
    pltpu.sync_copy(sp0.at[pl.ds(t0, chunk)], spv0)
    pltpu.sync_copy(sp1.at[pl.ds(t0, chunk)], spv1)
    pltpu.sync_copy(mt.at[n_i, 0, pl.ds(off, chunk)], wv0.at[pl.ds(0, chunk)])
    pltpu.sync_copy(mt.at[n_i, 1, pl.ds(off, chunk)], wv1.at[pl.ds(0, chunk)])
    ybufs = ((y0a, y1a, sem0a, sem1a, semxa), (y0b, y1b, sem0b, sem1b, semxb))
    xbufs = (xsa, xsb)
    nsub = chunk // sub

    def start(j, bank):
        y0, y1, s0, s1, sx = ybufs[bank]
        c0 = pltpu.async_copy(yg.at[spv0.at[pl.ds(j * sub, sub)]], y0, s0)
        c1 = pltpu.async_copy(yg.at[spv1.at[pl.ds(j * sub, sub)]], y1, s1)
        cx = pltpu.async_copy(xs.at[pl.ds(t0 + j * sub, sub)], xbufs[bank],
                              sx)
        return c0, c1, cx

    pend = start(0, 0)
    for j in range(nsub):
        bank = j % 2
        cur = pend
        if j + 1 < nsub:
            pend = start(j + 1, 1 - bank)
        for c in cur:
            c.wait()
        y0, y1 = ybufs[bank][0], ybufs[bank][1]
        xb = xbufs[bank]

        def body(t, _):
            ws0 = wv0[pl.ds(j * sub + t, _L)][0]
            ws1 = wv1[pl.ds(j * sub + t, _L)][0]
            for l in range(H // _L):
                sl = pl.ds(l * _L, _L)
                xb[t, sl] = xb[t, sl] + ws0 * y0[t, sl] + ws1 * y1[t, sl]
            return 0

        lax.fori_loop(0, sub, body, 0)
        pltpu.sync_copy(xb, out.at[pl.ds(t0 + j * sub, sub)])


def kernel(hidden_states, shared_gate_w, shared_up_w, shared_down_w,
           expert_gate_w, expert_up_w, expert_down_w, router_w):
    B, S, H = hidden_states.shape
    I = shared_gate_w.shape[0]
    T = B * S
    x2 = hidden_states.reshape(T, H)

    nt = T // _TB
    nb = (T * _K) // _BLK + _E
    pp = nb * _BLK

    a2_out = pl.pallas_call(
        _router_body,
        grid=(nt,),
        in_specs=[
            pl.BlockSpec((_TB, H), lambda t: (t, 0)),
            pl.BlockSpec((H, _E), lambda t: (0, 0)),
        ],
        out_specs=[
            pl.BlockSpec((1, 8, _TB), lambda t: (t, 0, 0)),
            pl.BlockSpec((8, 16), lambda t: (0, 0)),
            pl.BlockSpec((8, _NBE), lambda t: (0, 0)),
        ],
        out_shape=[
            jax.ShapeDtypeStruct((nt, 8, _TB), jnp.float32),
            jax.ShapeDtypeStruct((8, 16), jnp.int32),
            jax.ShapeDtypeStruct((8, _NBE), jnp.int32),
        ],
        scratch_shapes=[pltpu.VMEM((1, _E), jnp.float32)],
    )(x2, router_w.T)
    mt, pst, beo = a2_out
    used = pst[0, 8:9]
    be = jnp.concatenate([beo[0, :nb], used])

    xs = pl.pallas_call(
        _shared_body,
        grid=(nt,),
        in_specs=[
            pl.BlockSpec((_TB, H), lambda t: (t, 0)),
            pl.BlockSpec((I, H), lambda t: (0, 0)),
            pl.BlockSpec((I, H), lambda t: (0, 0)),
            pl.BlockSpec((H, I), lambda t: (0, 0)),
        ],
        out_specs=pl.BlockSpec((_TB, H), lambda t: (t, 0)),
        out_shape=jax.ShapeDtypeStruct((T, H), jnp.float32),
    )(x2, shared_gate_w, shared_up_w, shared_down_w)

    mesh = plsc.VectorSubcoreMesh(core_axis_name="c", subcore_axis_name="s")
    chunk = T // _NTILES
    sc_params = pltpu.CompilerParams(needs_layout_passes=False)
    sc_scatter = pl.kernel(
        _scatter_body,
        compiler_params=sc_params,
        out_type=[
            jax.ShapeDtypeStruct((pp, H), jnp.float32),
            jax.ShapeDtypeStruct((T,), jnp.int32),
            jax.ShapeDtypeStruct((T,), jnp.int32),
        ],
        mesh=mesh,
        scratch_types=[
            pltpu.VMEM((chunk,), jnp.float32),
            pltpu.VMEM((chunk,), jnp.float32),
            pltpu.VMEM((chunk,), jnp.float32),
            pltpu.VMEM((chunk,), jnp.float32),
            pltpu.VMEM((16,), jnp.int32),
            pltpu.VMEM((chunk,), jnp.int32),
            pltpu.VMEM((chunk,), jnp.int32),
            pltpu.VMEM((chunk // 32, 32), jnp.int32),
            pltpu.VMEM((chunk // 32, 32), jnp.int32),
            pltpu.VMEM((32, H), jnp.float32),
            pltpu.SemaphoreType.DMA,
        ],
    )
    xg, sp0, sp1 = sc_scatter(x2, mt, pst)

    yg = pl.pallas_call(
        _expert_body,
        grid_spec=pltpu.PrefetchScalarGridSpec(
            num_scalar_prefetch=1,
            grid=(nb,),
            in_specs=[
                pl.BlockSpec(
                    (_BLK, H),
                    lambda b, be_r: (jnp.minimum(b, be_r[nb] - 1), 0)),
                pl.BlockSpec((1, I, H), lambda b, be_r: (be_r[b], 0, 0)),
                pl.BlockSpec((1, I, H), lambda b, be_r: (be_r[b], 0, 0)),
                pl.BlockSpec((1, H, I), lambda b, be_r: (be_r[b], 0, 0)),
            ],
            out_specs=pl.BlockSpec(
                (_BLK, H),
                lambda b, be_r: (jnp.minimum(b, be_r[nb] - 1), 0)),
        ),
        out_shape=jax.ShapeDtypeStruct((pp, H), jnp.float32),
    )(be, xg, expert_gate_w, expert_up_w, expert_down_w)

    sub = 16
    sc_combine = pl.kernel(
        _combine_body,
        compiler_params=sc_params,
        out_type=jax.ShapeDtypeStruct((T, H), jnp.float32),
        mesh=mesh,
        scratch_types=[
            pltpu.VMEM((chunk,), jnp.int32),
            pltpu.VMEM((chunk,), jnp.int32),
            pltpu.VMEM((chunk + _L,), jnp.float32),
            pltpu.VMEM((chunk + _L,), jnp.float32),
            pltpu.VMEM((sub, H), jnp.float32),
            pltpu.VMEM((sub, H), jnp.float32),
            pltpu.VMEM((sub, H), jnp.float32),
            pltpu.VMEM((sub, H), jnp.float32),
            pltpu.VMEM((sub, H), jnp.float32),
            pltpu.VMEM((sub, H), jnp.float32),
            pltpu.SemaphoreType.DMA,
            pltpu.SemaphoreType.DMA,
            pltpu.SemaphoreType.DMA,
            pltpu.SemaphoreType.DMA,
            pltpu.SemaphoreType.DMA,
            pltpu.SemaphoreType.DMA,
        ],
    )
    out2 = sc_combine(xs, yg, sp0, sp1, mt)
    return out2.reshape(B, S, H)


# speedup vs baseline: 1.0814x; 1.0369x over previous
import jax
import jax.numpy as jnp
from jax import lax
from jax.experimental import pallas as pl
from jax.experimental.pallas import tpu as pltpu
from jax.experimental.pallas import tpu_sc as plsc

_E = 8
_K = 2
_TB = 512
_BLK = 512
_NTILES = 32
_L = 16
_NBE = 64


def _mm_t(a, b):
    return lax.dot_general(a, b, (((1,), (1,)), ((), ())),
                           preferred_element_type=jnp.float32)


def _mm_t16(a, b):
    return _mm_t(a.astype(jnp.bfloat16), b.astype(jnp.bfloat16))


def _shared_body(x_ref, sg_ref, su_ref, sd_ref, xs_ref):
    x = x_ref[...]
    g = _mm_t16(x, sg_ref[...])
    u = _mm_t16(x, su_ref[...])
    h = jax.nn.sigmoid(g) * u
    xs_ref[...] = x + _mm_t16(h, sd_ref[...])


def _router_body(x_ref, r_ref, mt_ref, pst_ref, beo_ref, carry_ref):
    t = pl.program_id(0)
    nt = pl.num_programs(0)

    @pl.when(t == 0)
    def _init():
        carry_ref[...] = jnp.zeros_like(carry_ref)

    x = x_ref[...]
    logits = jnp.dot(x, r_ref[...], preferred_element_type=jnp.float32)
    m = jnp.max(logits, axis=-1, keepdims=True)
    ex = jnp.exp(logits - m)
    sm = ex / jnp.sum(ex, axis=-1, keepdims=True)
    ids = jax.lax.broadcasted_iota(jnp.int32, sm.shape, 1)
    m1 = jnp.max(sm, axis=-1, keepdims=True)
    i1 = jnp.min(jnp.where(sm == m1, ids, _E), axis=-1, keepdims=True)
    s2 = jnp.where(ids == i1, -jnp.inf, sm)
    m2 = jnp.max(s2, axis=-1, keepdims=True)
    i2 = jnp.min(jnp.where(s2 == m2, ids, _E), axis=-1, keepdims=True)

    oh0 = (ids == i1).astype(jnp.float32)
    oh1 = (ids == i2).astype(jnp.float32)
    oh = oh0 + oh1
    row = jax.lax.broadcasted_iota(jnp.int32, (_TB, _TB), 0)
    col = jax.lax.broadcasted_iota(jnp.int32, (_TB, _TB), 1)
    tril = (row > col).astype(jnp.float32)
    c = jnp.dot(tril, oh, preferred_element_type=jnp.float32) + carry_ref[...]
    r0 = jnp.sum(c * oh0, axis=-1, keepdims=True)
    r1 = jnp.sum(c * oh1, axis=-1, keepdims=True)
    carry_new = carry_ref[...] + jnp.sum(oh, axis=0, keepdims=True)
    carry_ref[...] = carry_new

    lane = jax.lax.broadcasted_iota(jnp.int32, (_TB, 8), 1)
    meta = jnp.where(
        lane == 0, m1,
        jnp.where(lane == 1, m2,
                  jnp.where(lane == 2, i1.astype(jnp.float32),
                            jnp.where(lane == 3, i2.astype(jnp.float32),
                                      jnp.where(lane == 4, r0,
                                                jnp.where(lane == 5, r1,
                                                          0.0))))))
    eye = (row == col).astype(jnp.float32)
    mt_ref[...] = lax.dot_general(
        meta, eye, (((0,), (0,)), ((), ())),
        precision=lax.Precision.HIGHEST,
        preferred_element_type=jnp.float32)[None]

    @pl.when(t == nt - 1)
    def _finish():
        cntv = carry_new
        bc = jnp.floor((cntv + (_BLK - 1)) * (1.0 / _BLK))
        erow = jax.lax.broadcasted_iota(jnp.int32, (_E, _E), 0)
        ecol = jax.lax.broadcasted_iota(jnp.int32, (_E, _E), 1)
        lower = (erow <= ecol).astype(jnp.float32)
        cum = jnp.dot(bc, lower, preferred_element_type=jnp.float32)
        excl = cum - bc
        pstv = jnp.concatenate(
            [excl * _BLK, cum[:, 7:8], jnp.zeros((1, 7), jnp.float32)],
            axis=1)
        pst_ref[...] = jnp.broadcast_to(pstv, (8, 16)).astype(jnp.int32)
        bvec = jax.lax.broadcasted_iota(
            jnp.int32, (1, _NBE), 1).astype(jnp.float32)
        acc = jnp.zeros((1, _NBE), jnp.float32)
        for e in range(_E):
            acc = acc + (cum[0, e] <= bvec).astype(jnp.float32)
        acc = jnp.clip(acc, 0, _E - 1)
        beo_ref[...] = jnp.broadcast_to(acc, (8, _NBE)).astype(jnp.int32)


def _scatter_body(x2, mt, pst, xg, sp0, sp1,
                  iv0, iv1, rv0, rv1, psv, spf0, spf1, sp2d0, sp2d1,
                  xbuf, sem):
    chunk = 4096 // _NTILES
    nsub = chunk // 32
    w = lax.axis_index("s") * 2 + lax.axis_index("c")
    t0 = w * chunk
    n_i = t0 // _TB
    off = t0 % _TB
    pltpu.sync_copy(mt.at[n_i, 2, pl.ds(off, chunk)], iv0)
    pltpu.sync_copy(mt.at[n_i, 3, pl.ds(off, chunk)], iv1)
    pltpu.sync_copy(mt.at[n_i, 4, pl.ds(off, chunk)], rv0)
    pltpu.sync_copy(mt.at[n_i, 5, pl.ds(off, chunk)], rv1)
    pltpu.sync_copy(pst.at[0], psv)
    for l in range(chunk // _L):
        sl = pl.ds(l * _L, _L)
        i0v = iv0[sl].astype(jnp.int32)
        i1v = iv1[sl].astype(jnp.int32)
        st0 = plsc.load_gather(psv, [i0v])
        st1 = plsc.load_gather(psv, [i1v])
        spv0 = st0 + rv0[sl].astype(jnp.int32)
        spv1 = st1 + rv1[sl].astype(jnp.int32)
        spf0[sl] = spv0
        spf1[sl] = spv1
        sp2d0[l // 2, pl.ds((l % 2) * _L, _L)] = spv0
        sp2d1[l // 2, pl.ds((l % 2) * _L, _L)] = spv1
    pltpu.sync_copy(spf0, sp0.at[pl.ds(t0, chunk)])
    pltpu.sync_copy(spf1, sp1.at[pl.ds(t0, chunk)])
    for j in range(nsub):
        pltpu.sync_copy(x2.at[pl.ds(t0 + j * 32, 32)], xbuf)
        pltpu.async_copy(xbuf, xg.at[sp2d0.at[j]], sem).wait()
        pltpu.async_copy(xbuf, xg.at[sp2d1.at[j]], sem).wait()


def _expert_body(be_ref, xg_ref, eg_ref, eu_ref, ed_ref, yg_ref):
    b = pl.program_id(0)
    nb = pl.num_programs(0)

    @pl.when(b < be_ref[nb])
    def _compute():
        x = xg_ref[...]
        ge = _mm_t16(x, eg_ref[0])
        ue = _mm_t16(x, eu_ref[0])
        he = jax.nn.sigmoid(ge) * ue
        yg_ref[...] = _mm_t16(he, ed_ref[0])


def _combine_body(xs, yg, sp0, sp1, mt, out,
                  spv0, spv1, wv0, wv1, y0a, y1a, y0b, y1b, xsa, xsb,
                  sem0a, sem1a, sem0b, sem1b, semxa, semxb):
    chunk = 4096 // _NTILES
    sub = 16
    H = 1024
    w = lax.axis_index("s") * 2 + lax.axis_index("c")
    t0 = w * chunk
    n_i = t0 // _TB
    off = t0 % _TB
    pltpu.sync_copy(sp0.at[pl.ds(t0, chunk)], spv0)
    pltpu.sync_copy(sp1.at[pl.ds(t0, chunk)], spv1)
    pltpu.sync_copy(mt.at[n_i, 0, pl.ds(off, chunk)], wv0.at[pl.ds(0, chunk)])
    pltpu.sync_copy(mt.at[n_i, 1, pl.ds(off, chunk)], wv1.at[pl.ds(0, chunk)])
    ybufs = ((y0a, y1a, sem0a, sem1a, semxa), (y0b, y1b, sem0b, sem1b, semxb))
    xbufs = (xsa, xsb)
    nsub = chunk // sub

    def start(j, bank):
        y0, y1, s0, s1, sx = ybufs[bank]
        c0 = pltpu.async_copy(yg.at[spv0.at[pl.ds(j * sub, sub)]], y0, s0)
        c1 = pltpu.async_copy(yg.at[spv1.at[pl.ds(j * sub, sub)]], y1, s1)
        cx = pltpu.async_copy(xs.at[pl.ds(t0 + j * sub, sub)], xbufs[bank],
                              sx)
        return c0, c1, cx

    pend = start(0, 0)
    for j in range(nsub):
        bank = j % 2
        cur = pend
        if j + 1 < nsub:
            pend = start(j + 1, 1 - bank)
        for c in cur:
            c.wait()
        y0, y1 = ybufs[bank][0], ybufs[bank][1]
        xb = xbufs[bank]

        def body(t, _):
            ws0 = wv0[pl.ds(j * sub + t, _L)][0]
            ws1 = wv1[pl.ds(j * sub + t, _L)][0]
            for l in range(H // _L):
                sl = pl.ds(l * _L, _L)
                xb[t, sl] = xb[t, sl] + ws0 * y0[t, sl] + ws1 * y1[t, sl]
            return 0

        lax.fori_loop(0, sub, body, 0)
        pltpu.sync_copy(xb, out.at[pl.ds(t0 + j * sub, sub)])


def kernel(hidden_states, shared_gate_w, shared_up_w, shared_down_w,
           expert_gate_w, expert_up_w, expert_down_w, router_w):
    B, S, H = hidden_states.shape
    I = shared_gate_w.shape[0]
    T = B * S
    x2 = hidden_states.reshape(T, H)

    nt = T // _TB
    nb = (T * _K) // _BLK + _E
    pp = nb * _BLK

    a2_out = pl.pallas_call(
        _router_body,
        grid=(nt,),
        in_specs=[
            pl.BlockSpec((_TB, H), lambda t: (t, 0)),
            pl.BlockSpec((H, _E), lambda t: (0, 0)),
        ],
        out_specs=[
            pl.BlockSpec((1, 8, _TB), lambda t: (t, 0, 0)),
            pl.BlockSpec((8, 16), lambda t: (0, 0)),
            pl.BlockSpec((8, _NBE), lambda t: (0, 0)),
        ],
        out_shape=[
            jax.ShapeDtypeStruct((nt, 8, _TB), jnp.float32),
            jax.ShapeDtypeStruct((8, 16), jnp.int32),
            jax.ShapeDtypeStruct((8, _NBE), jnp.int32),
        ],
        scratch_shapes=[pltpu.VMEM((1, _E), jnp.float32)],
    )(x2, router_w.T)
    mt, pst, beo = a2_out
    used = pst[0, 8:9]
    be = jnp.concatenate([beo[0, :nb], used])

    xs = pl.pallas_call(
        _shared_body,
        grid=(nt,),
        in_specs=[
            pl.BlockSpec((_TB, H), lambda t: (t, 0)),
            pl.BlockSpec((I, H), lambda t: (0, 0)),
            pl.BlockSpec((I, H), lambda t: (0, 0)),
            pl.BlockSpec((H, I), lambda t: (0, 0)),
        ],
        out_specs=pl.BlockSpec((_TB, H), lambda t: (t, 0)),
        out_shape=jax.ShapeDtypeStruct((T, H), jnp.float32),
    )(x2, shared_gate_w, shared_up_w, shared_down_w)

    mesh = plsc.VectorSubcoreMesh(core_axis_name="c", subcore_axis_name="s")
    chunk = T // _NTILES
    sc_params = pltpu.CompilerParams(needs_layout_passes=False)
    sc_scatter = pl.kernel(
        _scatter_body,
        compiler_params=sc_params,
        out_type=[
            jax.ShapeDtypeStruct((pp, H), jnp.float32),
            jax.ShapeDtypeStruct((T,), jnp.int32),
            jax.ShapeDtypeStruct((T,), jnp.int32),
        ],
        mesh=mesh,
        scratch_types=[
            pltpu.VMEM((chunk,), jnp.float32),
            pltpu.VMEM((chunk,), jnp.float32),
            pltpu.VMEM((chunk,), jnp.float32),
            pltpu.VMEM((chunk,), jnp.float32),
            pltpu.VMEM((16,), jnp.int32),
            pltpu.VMEM((chunk,), jnp.int32),
            pltpu.VMEM((chunk,), jnp.int32),
            pltpu.VMEM((chunk // 32, 32), jnp.int32),
            pltpu.VMEM((chunk // 32, 32), jnp.int32),
            pltpu.VMEM((32, H), jnp.float32),
            pltpu.SemaphoreType.DMA,
        ],
    )
    xg, sp0, sp1 = sc_scatter(x2, mt, pst)

    yg = pl.pallas_call(
        _expert_body,
        grid_spec=pltpu.PrefetchScalarGridSpec(
            num_scalar_prefetch=1,
            grid=(nb,),
            in_specs=[
                pl.BlockSpec(
                    (_BLK, H),
                    lambda b, be_r: (jnp.minimum(b, be_r[nb] - 1), 0)),
                pl.BlockSpec((1, I, H), lambda b, be_r: (be_r[b], 0, 0)),
                pl.BlockSpec((1, I, H), lambda b, be_r: (be_r[b], 0, 0)),
                pl.BlockSpec((1, H, I), lambda b, be_r: (be_r[b], 0, 0)),
            ],
            out_specs=pl.BlockSpec(
                (_BLK, H),
                lambda b, be_r: (jnp.minimum(b, be_r[nb] - 1), 0)),
        ),
        out_shape=jax.ShapeDtypeStruct((pp, H), jnp.float32),
    )(be, xg, expert_gate_w, expert_up_w, expert_down_w)

    sub = 16
    sc_combine = pl.kernel(
        _combine_body,
        compiler_params=sc_params,
        out_type=jax.ShapeDtypeStruct((T, H), jnp.float32),
        mesh=mesh,
        scratch_types=[
            pltpu.VMEM((chunk,), jnp.int32),
            pltpu.VMEM((chunk,), jnp.int32),
            pltpu.VMEM((chunk + _L,), jnp.float32),
            pltpu.VMEM((chunk + _L,), jnp.float32),
            pltpu.VMEM((sub, H), jnp.float32),
            pltpu.VMEM((sub, H), jnp.float32),
            pltpu.VMEM((sub, H), jnp.float32),
            pltpu.VMEM((sub, H), jnp.float32),
            pltpu.VMEM((sub, H), jnp.float32),
            pltpu.VMEM((sub, H), jnp.float32),
            pltpu.SemaphoreType.DMA,
            pltpu.SemaphoreType.DMA,
            pltpu.SemaphoreType.DMA,
            pltpu.SemaphoreType.DMA,
            pltpu.SemaphoreType.DMA,
            pltpu.SemaphoreType.DMA,
        ],
    )
    out2 = sc_combine(xs, yg, sp0, sp1, mt)
    return out2.reshape(B, S, H)


# speedup vs baseline: 1.0918x; 1.0096x over previous
import jax
import jax.numpy as jnp
from jax import lax
from jax.experimental import pallas as pl
from jax.experimental.pallas import tpu as pltpu
from jax.experimental.pallas import tpu_sc as plsc

_E = 8
_K = 2
_TB = 512
_BLK = 1024
_NTILES = 32
_L = 16
_NBE = 64


def _mm_t(a, b):
    return lax.dot_general(a, b, (((1,), (1,)), ((), ())),
                           preferred_element_type=jnp.float32)


def _mm_t16(a, b):
    return _mm_t(a.astype(jnp.bfloat16), b.astype(jnp.bfloat16))


def _shared_body(x_ref, sg_ref, su_ref, sd_ref, xs_ref):
    x = x_ref[...]
    g = _mm_t16(x, sg_ref[...])
    u = _mm_t16(x, su_ref[...])
    h = jax.nn.sigmoid(g) * u
    xs_ref[...] = x + _mm_t16(h, sd_ref[...])


def _router_body(x_ref, r_ref, mt_ref, pst_ref, beo_ref, carry_ref):
    t = pl.program_id(0)
    nt = pl.num_programs(0)

    @pl.when(t == 0)
    def _init():
        carry_ref[...] = jnp.zeros_like(carry_ref)

    x = x_ref[...]
    logits = jnp.dot(x, r_ref[...], preferred_element_type=jnp.float32)
    m = jnp.max(logits, axis=-1, keepdims=True)
    ex = jnp.exp(logits - m)
    sm = ex / jnp.sum(ex, axis=-1, keepdims=True)
    ids = jax.lax.broadcasted_iota(jnp.int32, sm.shape, 1)
    m1 = jnp.max(sm, axis=-1, keepdims=True)
    i1 = jnp.min(jnp.where(sm == m1, ids, _E), axis=-1, keepdims=True)
    s2 = jnp.where(ids == i1, -jnp.inf, sm)
    m2 = jnp.max(s2, axis=-1, keepdims=True)
    i2 = jnp.min(jnp.where(s2 == m2, ids, _E), axis=-1, keepdims=True)

    oh0 = (ids == i1).astype(jnp.float32)
    oh1 = (ids == i2).astype(jnp.float32)
    oh = oh0 + oh1
    row = jax.lax.broadcasted_iota(jnp.int32, (_TB, _TB), 0)
    col = jax.lax.broadcasted_iota(jnp.int32, (_TB, _TB), 1)
    tril = (row > col).astype(jnp.float32)
    c = jnp.dot(tril, oh, preferred_element_type=jnp.float32) + carry_ref[...]
    r0 = jnp.sum(c * oh0, axis=-1, keepdims=True)
    r1 = jnp.sum(c * oh1, axis=-1, keepdims=True)
    carry_new = carry_ref[...] + jnp.sum(oh, axis=0, keepdims=True)
    carry_ref[...] = carry_new

    lane = jax.lax.broadcasted_iota(jnp.int32, (_TB, 8), 1)
    meta = jnp.where(
        lane == 0, m1,
        jnp.where(lane == 1, m2,
                  jnp.where(lane == 2, i1.astype(jnp.float32),
                            jnp.where(lane == 3, i2.astype(jnp.float32),
                                      jnp.where(lane == 4, r0,
                                                jnp.where(lane == 5, r1,
                                                          0.0))))))
    eye = (row == col).astype(jnp.float32)
    mt_ref[...] = lax.dot_general(
        meta, eye, (((0,), (0,)), ((), ())),
        precision=lax.Precision.HIGHEST,
        preferred_element_type=jnp.float32)[None]

    @pl.when(t == nt - 1)
    def _finish():
        cntv = carry_new
        bc = jnp.floor((cntv + (_BLK - 1)) * (1.0 / _BLK))
        erow = jax.lax.broadcasted_iota(jnp.int32, (_E, _E), 0)
        ecol = jax.lax.broadcasted_iota(jnp.int32, (_E, _E), 1)
        lower = (erow <= ecol).astype(jnp.float32)
        cum = jnp.dot(bc, lower, preferred_element_type=jnp.float32)
        excl = cum - bc
        pstv = jnp.concatenate(
            [excl * _BLK, cum[:, 7:8], jnp.zeros((1, 7), jnp.float32)],
            axis=1)
        pst_ref[...] = jnp.broadcast_to(pstv, (8, 16)).astype(jnp.int32)
        bvec = jax.lax.broadcasted_iota(
            jnp.int32, (1, _NBE), 1).astype(jnp.float32)
        acc = jnp.zeros((1, _NBE), jnp.float32)
        for e in range(_E):
            acc = acc + (cum[0, e] <= bvec).astype(jnp.float32)
        acc = jnp.clip(acc, 0, _E - 1)
        beo_ref[...] = jnp.broadcast_to(acc, (8, _NBE)).astype(jnp.int32)


def _scatter_body(x2, mt, pst, xg, sp0, sp1,
                  iv0, iv1, rv0, rv1, psv, spf0, spf1, sp2d0, sp2d1,
                  xbuf, sem):
    chunk = 4096 // _NTILES
    nsub = chunk // 32
    w = lax.axis_index("s") * 2 + lax.axis_index("c")
    t0 = w * chunk
    n_i = t0 // _TB
    off = t0 % _TB
    pltpu.sync_copy(mt.at[n_i, 2, pl.ds(off, chunk)], iv0)
    pltpu.sync_copy(mt.at[n_i, 3, pl.ds(off, chunk)], iv1)
    pltpu.sync_copy(mt.at[n_i, 4, pl.ds(off, chunk)], rv0)
    pltpu.sync_copy(mt.at[n_i, 5, pl.ds(off, chunk)], rv1)
    pltpu.sync_copy(pst.at[0], psv)
    for l in range(chunk // _L):
        sl = pl.ds(l * _L, _L)
        i0v = iv0[sl].astype(jnp.int32)
        i1v = iv1[sl].astype(jnp.int32)
        st0 = plsc.load_gather(psv, [i0v])
        st1 = plsc.load_gather(psv, [i1v])
        spv0 = st0 + rv0[sl].astype(jnp.int32)
        spv1 = st1 + rv1[sl].astype(jnp.int32)
        spf0[sl] = spv0
        spf1[sl] = spv1
        sp2d0[l // 2, pl.ds((l % 2) * _L, _L)] = spv0
        sp2d1[l // 2, pl.ds((l % 2) * _L, _L)] = spv1
    pltpu.sync_copy(spf0, sp0.at[pl.ds(t0, chunk)])
    pltpu.sync_copy(spf1, sp1.at[pl.ds(t0, chunk)])
    for j in range(nsub):
        pltpu.sync_copy(x2.at[pl.ds(t0 + j * 32, 32)], xbuf)
        pltpu.async_copy(xbuf, xg.at[sp2d0.at[j]], sem).wait()
        pltpu.async_copy(xbuf, xg.at[sp2d1.at[j]], sem).wait()


def _expert_body(be_ref, xg_ref, eg_ref, eu_ref, ed_ref, yg_ref):
    b = pl.program_id(0)
    nb = pl.num_programs(0)

    @pl.when(b < be_ref[nb])
    def _compute():
        x = xg_ref[...]
        ge = _mm_t16(x, eg_ref[0])
        ue = _mm_t16(x, eu_ref[0])
        he = jax.nn.sigmoid(ge) * ue
        yg_ref[...] = _mm_t16(he, ed_ref[0])


def _combine_body(xs, yg, sp0, sp1, mt, out,
                  spv0, spv1, wv0, wv1, y0a, y1a, y0b, y1b, xsa, xsb,
                  sem0a, sem1a, sem0b, sem1b, semxa, semxb):
    chunk = 4096 // _NTILES
    sub = 16
    H = 1024
    w = lax.axis_index("s") * 2 + lax.axis_index("c")
    t0 = w * chunk
    n_i = t0 // _TB
    off = t0 % _TB
    pltpu.sync_copy(sp0.at[pl.ds(t0, chunk)], spv0)
    pltpu.sync_copy(sp1.at[pl.ds(t0, chunk)], spv1)
    pltpu.sync_copy(mt.at[n_i, 0, pl.ds(off, chunk)], wv0.at[pl.ds(0, chunk)])
    pltpu.sync_copy(mt.at[n_i, 1, pl.ds(off, chunk)], wv1.at[pl.ds(0, chunk)])
    ybufs = ((y0a, y1a, sem0a, sem1a, semxa), (y0b, y1b, sem0b, sem1b, semxb))
    xbufs = (xsa, xsb)
    nsub = chunk // sub

    def start(j, bank):
        y0, y1, s0, s1, sx = ybufs[bank]
        c0 = pltpu.async_copy(yg.at[spv0.at[pl.ds(j * sub, sub)]], y0, s0)
        c1 = pltpu.async_copy(yg.at[spv1.at[pl.ds(j * sub, sub)]], y1, s1)
        cx = pltpu.async_copy(xs.at[pl.ds(t0 + j * sub, sub)], xbufs[bank],
                              sx)
        return c0, c1, cx

    pend = start(0, 0)
    for j in range(nsub):
        bank = j % 2
        cur = pend
        if j + 1 < nsub:
            pend = start(j + 1, 1 - bank)
        for c in cur:
            c.wait()
        y0, y1 = ybufs[bank][0], ybufs[bank][1]
        xb = xbufs[bank]

        def body(t, _):
            ws0 = wv0[pl.ds(j * sub + t, _L)][0]
            ws1 = wv1[pl.ds(j * sub + t, _L)][0]
            for l in range(H // _L):
                sl = pl.ds(l * _L, _L)
                xb[t, sl] = xb[t, sl] + ws0 * y0[t, sl] + ws1 * y1[t, sl]
            return 0

        lax.fori_loop(0, sub, body, 0)
        pltpu.sync_copy(xb, out.at[pl.ds(t0 + j * sub, sub)])


def kernel(hidden_states, shared_gate_w, shared_up_w, shared_down_w,
           expert_gate_w, expert_up_w, expert_down_w, router_w):
    B, S, H = hidden_states.shape
    I = shared_gate_w.shape[0]
    T = B * S
    x2 = hidden_states.reshape(T, H)

    nt = T // _TB
    nb = (T * _K) // _BLK + _E
    pp = nb * _BLK

    a2_out = pl.pallas_call(
        _router_body,
        grid=(nt,),
        in_specs=[
            pl.BlockSpec((_TB, H), lambda t: (t, 0)),
            pl.BlockSpec((H, _E), lambda t: (0, 0)),
        ],
        out_specs=[
            pl.BlockSpec((1, 8, _TB), lambda t: (t, 0, 0)),
            pl.BlockSpec((8, 16), lambda t: (0, 0)),
            pl.BlockSpec((8, _NBE), lambda t: (0, 0)),
        ],
        out_shape=[
            jax.ShapeDtypeStruct((nt, 8, _TB), jnp.float32),
            jax.ShapeDtypeStruct((8, 16), jnp.int32),
            jax.ShapeDtypeStruct((8, _NBE), jnp.int32),
        ],
        scratch_shapes=[pltpu.VMEM((1, _E), jnp.float32)],
    )(x2, router_w.T)
    mt, pst, beo = a2_out
    used = pst[0, 8:9]
    be = jnp.concatenate([beo[0, :nb], used])

    xs = pl.pallas_call(
        _shared_body,
        grid=(nt,),
        in_specs=[
            pl.BlockSpec((_TB, H), lambda t: (t, 0)),
            pl.BlockSpec((I, H), lambda t: (0, 0)),
            pl.BlockSpec((I, H), lambda t: (0, 0)),
            pl.BlockSpec((H, I), lambda t: (0, 0)),
        ],
        out_specs=pl.BlockSpec((_TB, H), lambda t: (t, 0)),
        out_shape=jax.ShapeDtypeStruct((T, H), jnp.float32),
    )(x2, shared_gate_w, shared_up_w, shared_down_w)

    mesh = plsc.VectorSubcoreMesh(core_axis_name="c", subcore_axis_name="s")
    chunk = T // _NTILES
    sc_params = pltpu.CompilerParams(needs_layout_passes=False)
    sc_scatter = pl.kernel(
        _scatter_body,
        compiler_params=sc_params,
        out_type=[
            jax.ShapeDtypeStruct((pp, H), jnp.float32),
            jax.ShapeDtypeStruct((T,), jnp.int32),
            jax.ShapeDtypeStruct((T,), jnp.int32),
        ],
        mesh=mesh,
        scratch_types=[
            pltpu.VMEM((chunk,), jnp.float32),
            pltpu.VMEM((chunk,), jnp.float32),
            pltpu.VMEM((chunk,), jnp.float32),
            pltpu.VMEM((chunk,), jnp.float32),
            pltpu.VMEM((16,), jnp.int32),
            pltpu.VMEM((chunk,), jnp.int32),
            pltpu.VMEM((chunk,), jnp.int32),
            pltpu.VMEM((chunk // 32, 32), jnp.int32),
            pltpu.VMEM((chunk // 32, 32), jnp.int32),
            pltpu.VMEM((32, H), jnp.float32),
            pltpu.SemaphoreType.DMA,
        ],
    )
    xg, sp0, sp1 = sc_scatter(x2, mt, pst)

    yg = pl.pallas_call(
        _expert_body,
        grid_spec=pltpu.PrefetchScalarGridSpec(
            num_scalar_prefetch=1,
            grid=(nb,),
            in_specs=[
                pl.BlockSpec(
                    (_BLK, H),
                    lambda b, be_r: (jnp.minimum(b, be_r[nb] - 1), 0)),
                pl.BlockSpec((1, I, H), lambda b, be_r: (be_r[b], 0, 0)),
                pl.BlockSpec((1, I, H), lambda b, be_r: (be_r[b], 0, 0)),
                pl.BlockSpec((1, H, I), lambda b, be_r: (be_r[b], 0, 0)),
            ],
            out_specs=pl.BlockSpec(
                (_BLK, H),
                lambda b, be_r: (jnp.minimum(b, be_r[nb] - 1), 0)),
        ),
        out_shape=jax.ShapeDtypeStruct((pp, H), jnp.float32),
    )(be, xg, expert_gate_w, expert_up_w, expert_down_w)

    sub = 16
    sc_combine = pl.kernel(
        _combine_body,
        compiler_params=sc_params,
        out_type=jax.ShapeDtypeStruct((T, H), jnp.float32),
        mesh=mesh,
        scratch_types=[
            pltpu.VMEM((chunk,), jnp.int32),
            pltpu.VMEM((chunk,), jnp.int32),
            pltpu.VMEM((chunk + _L,), jnp.float32),
            pltpu.VMEM((chunk + _L,), jnp.float32),
            pltpu.VMEM((sub, H), jnp.float32),
            pltpu.VMEM((sub, H), jnp.float32),
            pltpu.VMEM((sub, H), jnp.float32),
            pltpu.VMEM((sub, H), jnp.float32),
            pltpu.VMEM((sub, H), jnp.float32),
            pltpu.VMEM((sub, H), jnp.float32),
            pltpu.SemaphoreType.DMA,
            pltpu.SemaphoreType.DMA,
            pltpu.SemaphoreType.DMA,
            pltpu.SemaphoreType.DMA,
            pltpu.SemaphoreType.DMA,
            pltpu.SemaphoreType.DMA,
        ],
    )
    out2 = sc_combine(xs, yg, sp0, sp1, mt)
    return out2.reshape(B, S, H)


# speedup vs baseline: 1.1016x; 1.0090x over previous
import jax
import jax.numpy as jnp
from jax import lax
from jax.experimental import pallas as pl
from jax.experimental.pallas import tpu as pltpu
from jax.experimental.pallas import tpu_sc as plsc

_E = 8
_K = 2
_TB = 512
_BLK = 1024
_NTILES = 32
_L = 16
_NBE = 64


def _mm_t(a, b):
    return lax.dot_general(a, b, (((1,), (1,)), ((), ())),
                           preferred_element_type=jnp.float32)


def _mm_t16(a, b):
    return _mm_t(a.astype(jnp.bfloat16), b.astype(jnp.bfloat16))


def _shared_body(x_ref, sg_ref, su_ref, sd_ref, xs_ref):
    x = x_ref[...]
    g = _mm_t16(x, sg_ref[...])
    u = _mm_t16(x, su_ref[...])
    h = jax.nn.sigmoid(g) * u
    xs_ref[...] = x + _mm_t16(h, sd_ref[...])


def _router_body(x_ref, r_ref, mt_ref, pst_ref, beo_ref, carry_ref):
    t = pl.program_id(0)
    nt = pl.num_programs(0)

    @pl.when(t == 0)
    def _init():
        carry_ref[...] = jnp.zeros_like(carry_ref)

    x = x_ref[...]
    logits = jnp.dot(x, r_ref[...], preferred_element_type=jnp.float32)
    m = jnp.max(logits, axis=-1, keepdims=True)
    ex = jnp.exp(logits - m)
    sm = ex / jnp.sum(ex, axis=-1, keepdims=True)
    ids = jax.lax.broadcasted_iota(jnp.int32, sm.shape, 1)
    m1 = jnp.max(sm, axis=-1, keepdims=True)
    i1 = jnp.min(jnp.where(sm == m1, ids, _E), axis=-1, keepdims=True)
    s2 = jnp.where(ids == i1, -jnp.inf, sm)
    m2 = jnp.max(s2, axis=-1, keepdims=True)
    i2 = jnp.min(jnp.where(s2 == m2, ids, _E), axis=-1, keepdims=True)

    oh0 = (ids == i1).astype(jnp.float32)
    oh1 = (ids == i2).astype(jnp.float32)
    oh = oh0 + oh1
    row = jax.lax.broadcasted_iota(jnp.int32, (_TB, _TB), 0)
    col = jax.lax.broadcasted_iota(jnp.int32, (_TB, _TB), 1)
    tril = (row > col).astype(jnp.float32)
    c = jnp.dot(tril, oh, preferred_element_type=jnp.float32) + carry_ref[...]
    r0 = jnp.sum(c * oh0, axis=-1, keepdims=True)
    r1 = jnp.sum(c * oh1, axis=-1, keepdims=True)
    carry_new = carry_ref[...] + jnp.sum(oh, axis=0, keepdims=True)
    carry_ref[...] = carry_new

    lane = jax.lax.broadcasted_iota(jnp.int32, (_TB, 8), 1)
    meta = jnp.where(
        lane == 0, m1,
        jnp.where(lane == 1, m2,
                  jnp.where(lane == 2, i1.astype(jnp.float32),
                            jnp.where(lane == 3, i2.astype(jnp.float32),
                                      jnp.where(lane == 4, r0,
                                                jnp.where(lane == 5, r1,
                                                          0.0))))))
    eye = (row == col).astype(jnp.float32)
    mt_ref[...] = lax.dot_general(
        meta, eye, (((0,), (0,)), ((), ())),
        precision=lax.Precision.HIGHEST,
        preferred_element_type=jnp.float32)[None]

    @pl.when(t == nt - 1)
    def _finish():
        cntv = carry_new
        bc = jnp.floor((cntv + (_BLK - 1)) * (1.0 / _BLK))
        erow = jax.lax.broadcasted_iota(jnp.int32, (_E, _E), 0)
        ecol = jax.lax.broadcasted_iota(jnp.int32, (_E, _E), 1)
        lower = (erow <= ecol).astype(jnp.float32)
        cum = jnp.dot(bc, lower, preferred_element_type=jnp.float32)
        excl = cum - bc
        pstv = jnp.concatenate(
            [excl * _BLK, cum[:, 7:8], jnp.zeros((1, 7), jnp.float32)],
            axis=1)
        pst_ref[...] = jnp.broadcast_to(pstv, (8, 16)).astype(jnp.int32)
        bvec = jax.lax.broadcasted_iota(
            jnp.int32, (1, _NBE), 1).astype(jnp.float32)
        acc = jnp.zeros((1, _NBE), jnp.float32)
        for e in range(_E):
            acc = acc + (cum[0, e] <= bvec).astype(jnp.float32)
        acc = jnp.clip(acc, 0, _E - 1)
        beo_ref[...] = jnp.broadcast_to(acc, (8, _NBE)).astype(jnp.int32)


def _scatter_body(x2, mt, pst, xg, sp0, sp1,
                  iv0, iv1, rv0, rv1, psv, spf0, spf1, sp2d0, sp2d1,
                  xbufa, xbufb, s0a, s1a, s0b, s1b):
    chunk = 4096 // _NTILES
    nsub = chunk // 32
    w = lax.axis_index("s") * 2 + lax.axis_index("c")
    t0 = w * chunk
    n_i = t0 // _TB
    off = t0 % _TB
    pltpu.sync_copy(mt.at[n_i, 2, pl.ds(off, chunk)], iv0)
    pltpu.sync_copy(mt.at[n_i, 3, pl.ds(off, chunk)], iv1)
    pltpu.sync_copy(mt.at[n_i, 4, pl.ds(off, chunk)], rv0)
    pltpu.sync_copy(mt.at[n_i, 5, pl.ds(off, chunk)], rv1)
    pltpu.sync_copy(pst.at[0], psv)
    for l in range(chunk // _L):
        sl = pl.ds(l * _L, _L)
        i0v = iv0[sl].astype(jnp.int32)
        i1v = iv1[sl].astype(jnp.int32)
        st0 = plsc.load_gather(psv, [i0v])
        st1 = plsc.load_gather(psv, [i1v])
        spv0 = st0 + rv0[sl].astype(jnp.int32)
        spv1 = st1 + rv1[sl].astype(jnp.int32)
        spf0[sl] = spv0
        spf1[sl] = spv1
        sp2d0[l // 2, pl.ds((l % 2) * _L, _L)] = spv0
        sp2d1[l // 2, pl.ds((l % 2) * _L, _L)] = spv1
    pltpu.sync_copy(spf0, sp0.at[pl.ds(t0, chunk)])
    pltpu.sync_copy(spf1, sp1.at[pl.ds(t0, chunk)])
    banks = ((xbufa, s0a, s1a), (xbufb, s0b, s1b))
    pend = [None, None]
    for j in range(nsub):
        xbuf, s0, s1 = banks[j % 2]
        if pend[j % 2] is not None:
            for c in pend[j % 2]:
                c.wait()
        pltpu.sync_copy(x2.at[pl.ds(t0 + j * 32, 32)], xbuf)
        c0 = pltpu.async_copy(xbuf, xg.at[sp2d0.at[j]], s0)
        c1 = pltpu.async_copy(xbuf, xg.at[sp2d1.at[j]], s1)
        pend[j % 2] = (c0, c1)
    for p in pend:
        if p is not None:
            for c in p:
                c.wait()


def _expert_body(be_ref, xg_ref, eg_ref, eu_ref, ed_ref, yg_ref):
    b = pl.program_id(0)
    nb = pl.num_programs(0)

    @pl.when(b < be_ref[nb])
    def _compute():
        x = xg_ref[...]
        ge = _mm_t16(x, eg_ref[0])
        ue = _mm_t16(x, eu_ref[0])
        he = jax.nn.sigmoid(ge) * ue
        yg_ref[...] = _mm_t16(he, ed_ref[0])


def _combine_body(xs, yg, sp0, sp1, mt, out,
                  spv0, spv1, wv0, wv1, y0a, y1a, y0b, y1b, xsa, xsb,
                  sem0a, sem1a, sem0b, sem1b, semxa, semxb):
    chunk = 4096 // _NTILES
    sub = 16
    H = 1024
    w = lax.axis_index("s") * 2 + lax.axis_index("c")
    t0 = w * chunk
    n_i = t0 // _TB
    off = t0 % _TB
    pltpu.sync_copy(sp0.at[pl.ds(t0, chunk)], spv0)
    pltpu.sync_copy(sp1.at[pl.ds(t0, chunk)], spv1)
    pltpu.sync_copy(mt.at[n_i, 0, pl.ds(off, chunk)], wv0.at[pl.ds(0, chunk)])
    pltpu.sync_copy(mt.at[n_i, 1, pl.ds(off, chunk)], wv1.at[pl.ds(0, chunk)])
    ybufs = ((y0a, y1a, sem0a, sem1a, semxa), (y0b, y1b, sem0b, sem1b, semxb))
    xbufs = (xsa, xsb)
    nsub = chunk // sub

    def start(j, bank):
        y0, y1, s0, s1, sx = ybufs[bank]
        c0 = pltpu.async_copy(yg.at[spv0.at[pl.ds(j * sub, sub)]], y0, s0)
        c1 = pltpu.async_copy(yg.at[spv1.at[pl.ds(j * sub, sub)]], y1, s1)
        cx = pltpu.async_copy(xs.at[pl.ds(t0 + j * sub, sub)], xbufs[bank],
                              sx)
        return c0, c1, cx

    pend = start(0, 0)
    for j in range(nsub):
        bank = j % 2
        cur = pend
        if j + 1 < nsub:
            pend = start(j + 1, 1 - bank)
        for c in cur:
            c.wait()
        y0, y1 = ybufs[bank][0], ybufs[bank][1]
        xb = xbufs[bank]

        def body(t, _):
            ws0 = wv0[pl.ds(j * sub + t, _L)][0]
            ws1 = wv1[pl.ds(j * sub + t, _L)][0]
            for l in range(H // _L):
                sl = pl.ds(l * _L, _L)
                xb[t, sl] = xb[t, sl] + ws0 * y0[t, sl] + ws1 * y1[t, sl]
            return 0

        lax.fori_loop(0, sub, body, 0)
        pltpu.sync_copy(xb, out.at[pl.ds(t0 + j * sub, sub)])


def kernel(hidden_states, shared_gate_w, shared_up_w, shared_down_w,
           expert_gate_w, expert_up_w, expert_down_w, router_w):
    B, S, H = hidden_states.shape
    I = shared_gate_w.shape[0]
    T = B * S
    x2 = hidden_states.reshape(T, H)

    nt = T // _TB
    nb = (T * _K) // _BLK + _E
    pp = nb * _BLK

    a2_out = pl.pallas_call(
        _router_body,
        grid=(nt,),
        in_specs=[
            pl.BlockSpec((_TB, H), lambda t: (t, 0)),
            pl.BlockSpec((H, _E), lambda t: (0, 0)),
        ],
        out_specs=[
            pl.BlockSpec((1, 8, _TB), lambda t: (t, 0, 0)),
            pl.BlockSpec((8, 16), lambda t: (0, 0)),
            pl.BlockSpec((8, _NBE), lambda t: (0, 0)),
        ],
        out_shape=[
            jax.ShapeDtypeStruct((nt, 8, _TB), jnp.float32),
            jax.ShapeDtypeStruct((8, 16), jnp.int32),
            jax.ShapeDtypeStruct((8, _NBE), jnp.int32),
        ],
        scratch_shapes=[pltpu.VMEM((1, _E), jnp.float32)],
    )(x2, router_w.T)
    mt, pst, beo = a2_out
    used = pst[0, 8:9]
    be = jnp.concatenate([beo[0, :nb], used])

    xs = pl.pallas_call(
        _shared_body,
        grid=(nt,),
        in_specs=[
            pl.BlockSpec((_TB, H), lambda t: (t, 0)),
            pl.BlockSpec((I, H), lambda t: (0, 0)),
            pl.BlockSpec((I, H), lambda t: (0, 0)),
            pl.BlockSpec((H, I), lambda t: (0, 0)),
        ],
        out_specs=pl.BlockSpec((_TB, H), lambda t: (t, 0)),
        out_shape=jax.ShapeDtypeStruct((T, H), jnp.float32),
    )(x2, shared_gate_w, shared_up_w, shared_down_w)

    mesh = plsc.VectorSubcoreMesh(core_axis_name="c", subcore_axis_name="s")
    chunk = T // _NTILES
    sc_params = pltpu.CompilerParams(needs_layout_passes=False)
    sc_scatter = pl.kernel(
        _scatter_body,
        compiler_params=sc_params,
        out_type=[
            jax.ShapeDtypeStruct((pp, H), jnp.float32),
            jax.ShapeDtypeStruct((T,), jnp.int32),
            jax.ShapeDtypeStruct((T,), jnp.int32),
        ],
        mesh=mesh,
        scratch_types=[
            pltpu.VMEM((chunk,), jnp.float32),
            pltpu.VMEM((chunk,), jnp.float32),
            pltpu.VMEM((chunk,), jnp.float32),
            pltpu.VMEM((chunk,), jnp.float32),
            pltpu.VMEM((16,), jnp.int32),
            pltpu.VMEM((chunk,), jnp.int32),
            pltpu.VMEM((chunk,), jnp.int32),
            pltpu.VMEM((chunk // 32, 32), jnp.int32),
            pltpu.VMEM((chunk // 32, 32), jnp.int32),
            pltpu.VMEM((32, H), jnp.float32),
            pltpu.VMEM((32, H), jnp.float32),
            pltpu.SemaphoreType.DMA,
            pltpu.SemaphoreType.DMA,
            pltpu.SemaphoreType.DMA,
            pltpu.SemaphoreType.DMA,
        ],
    )
    xg, sp0, sp1 = sc_scatter(x2, mt, pst)

    yg = pl.pallas_call(
        _expert_body,
        grid_spec=pltpu.PrefetchScalarGridSpec(
            num_scalar_prefetch=1,
            grid=(nb,),
            in_specs=[
                pl.BlockSpec(
                    (_BLK, H),
                    lambda b, be_r: (jnp.minimum(b, be_r[nb] - 1), 0)),
                pl.BlockSpec((1, I, H), lambda b, be_r: (be_r[b], 0, 0)),
                pl.BlockSpec((1, I, H), lambda b, be_r: (be_r[b], 0, 0)),
                pl.BlockSpec((1, H, I), lambda b, be_r: (be_r[b], 0, 0)),
            ],
            out_specs=pl.BlockSpec(
                (_BLK, H),
                lambda b, be_r: (jnp.minimum(b, be_r[nb] - 1), 0)),
        ),
        out_shape=jax.ShapeDtypeStruct((pp, H), jnp.float32),
    )(be, xg, expert_gate_w, expert_up_w, expert_down_w)

    sub = 16
    sc_combine = pl.kernel(
        _combine_body,
        compiler_params=sc_params,
        out_type=jax.ShapeDtypeStruct((T, H), jnp.float32),
        mesh=mesh,
        scratch_types=[
            pltpu.VMEM((chunk,), jnp.int32),
            pltpu.VMEM((chunk,), jnp.int32),
            pltpu.VMEM((chunk + _L,), jnp.float32),
            pltpu.VMEM((chunk + _L,), jnp.float32),
            pltpu.VMEM((sub, H), jnp.float32),
            pltpu.VMEM((sub, H), jnp.float32),
            pltpu.VMEM((sub, H), jnp.float32),
            pltpu.VMEM((sub, H), jnp.float32),
            pltpu.VMEM((sub, H), jnp.float32),
            pltpu.VMEM((sub, H), jnp.float32),
            pltpu.SemaphoreType.DMA,
            pltpu.SemaphoreType.DMA,
            pltpu.SemaphoreType.DMA,
            pltpu.SemaphoreType.DMA,
            pltpu.SemaphoreType.DMA,
            pltpu.SemaphoreType.DMA,
        ],
    )
    out2 = sc_combine(xs, yg, sp0, sp1, mt)
    return out2.reshape(B, S, H)
